# baseline (speedup 1.0000x reference)
import functools

import jax
import jax.numpy as jnp
from jax import lax
from jax.experimental import pallas as pl
from jax.experimental.pallas import tpu as pltpu
from jax.experimental.pallas import tpu_sc as plsc

F32 = jnp.float32

NU = 100000
NI = 100000
NF = 16
NA = 8
NT = 32
BATCH = 16384
TW = NA * NT

L_U = 0.01
L_I = 0.01
L_A = 0.001
L_B = 0.01

NC = 2
NS = 16
NW = NC * NS
BPW = BATCH // NW
CH = 16
NCH = BPW // CH


def _sc_theta_body(uids, iids, th, ps, out_th, out_ps,
                   uidx, iidx, thbuf, psbuf, sg0, sg1, sw0, sw1):
    wid = lax.axis_index("s") * NC + lax.axis_index("c")
    base = wid * BPW
    pltpu.sync_copy(uids.at[pl.ds(base, BPW)], uidx)
    pltpu.sync_copy(iids.at[pl.ds(base, BPW)], iidx)

    gsem = (sg0, sg1)
    wsem = (sw0, sw1)
    gp = [None, None]
    wp = [None, None]

    def issue(c):
        b = c % 2
        if wp[b] is not None:
            for cp in wp[b]:
                cp.wait()
            wp[b] = None
        uv = uidx[pl.ds(c * CH, CH)]
        iv = iidx[pl.ds(c * CH, CH)]
        for l in range(CH):
            pltpu.async_copy(th.at[uv[l]], thbuf.at[b, l], gsem[b])
            pltpu.async_copy(ps.at[iv[l]], psbuf.at[b, l], gsem[b])
        gp[b] = [pltpu.make_async_copy(th.at[pl.ds(0, CH)], thbuf.at[b],
                                       gsem[b]),
                 pltpu.make_async_copy(ps.at[pl.ds(0, CH)], psbuf.at[b],
                                       gsem[b])]

    issue(0)
    for c in range(NCH):
        b = c % 2
        if c + 1 < NCH:
            issue(c + 1)
        for cp in gp[b]:
            cp.wait()
        osl = pl.ds(base + c * CH, CH)
        wp[b] = [pltpu.async_copy(thbuf.at[b], out_th.at[osl], wsem[b]),
                 pltpu.async_copy(psbuf.at[b], out_ps.at[osl], wsem[b])]
    for b in range(2):
        if wp[b] is not None:
            for cp in wp[b]:
                cp.wait()


@functools.cache
def _sc_theta():
    return pl.kernel(
        _sc_theta_body,
        out_type=[
            jax.ShapeDtypeStruct((BATCH, NA, NT), F32),
            jax.ShapeDtypeStruct((BATCH, NA, NT), F32),
        ],
        mesh=plsc.VectorSubcoreMesh(core_axis_name="c", subcore_axis_name="s"),
        scratch_types=[
            pltpu.VMEM((BPW,), jnp.int32),
            pltpu.VMEM((BPW,), jnp.int32),
            pltpu.VMEM((2, CH, NA, NT), F32),
            pltpu.VMEM((2, CH, NA, NT), F32),
            pltpu.SemaphoreType.DMA,
            pltpu.SemaphoreType.DMA,
            pltpu.SemaphoreType.DMA,
            pltpu.SemaphoreType.DMA,
        ],
    )


def _sc_small_body(uids, iids, ut, it, usm, ism,
                   out_u, out_i, out_us, out_is,
                   uidx, iidx, ubuf, ibuf, usbuf, isbuf, sem):
    wid = lax.axis_index("s") * NC + lax.axis_index("c")
    base = wid * BPW
    pltpu.sync_copy(uids.at[pl.ds(base, BPW)], uidx)
    pltpu.sync_copy(iids.at[pl.ds(base, BPW)], iidx)
    cps = [pltpu.async_copy(ut.at[uidx], ubuf, sem),
           pltpu.async_copy(it.at[iidx], ibuf, sem),
           pltpu.async_copy(usm.at[uidx], usbuf, sem),
           pltpu.async_copy(ism.at[iidx], isbuf, sem)]
    for cp in cps:
        cp.wait()
    osl = pl.ds(base, BPW)
    pltpu.sync_copy(ubuf, out_u.at[osl])
    pltpu.sync_copy(ibuf, out_i.at[osl])
    pltpu.sync_copy(usbuf, out_us.at[osl])
    pltpu.sync_copy(isbuf, out_is.at[osl])


@functools.cache
def _sc_small():
    return pl.kernel(
        _sc_small_body,
        out_type=[
            jax.ShapeDtypeStruct((BATCH, NF), F32),
            jax.ShapeDtypeStruct((BATCH, NF), F32),
            jax.ShapeDtypeStruct((BATCH, 16), F32),
            jax.ShapeDtypeStruct((BATCH, 16), F32),
        ],
        mesh=plsc.VectorSubcoreMesh(core_axis_name="c", subcore_axis_name="s"),
        compiler_params=pltpu.CompilerParams(use_tc_tiling_on_sc=False),
        scratch_types=[
            pltpu.VMEM((BPW,), jnp.int32),
            pltpu.VMEM((BPW,), jnp.int32),
            pltpu.VMEM((BPW, NF), F32),
            pltpu.VMEM((BPW, NF), F32),
            pltpu.VMEM((BPW, 16), F32),
            pltpu.VMEM((BPW, 16), F32),
            pltpu.SemaphoreType.DMA,
        ],
    )


BLK = 1024
NBLK = BATCH // BLK


def _tc_math_body(th_ref, ps_ref, u_ref, i_ref, us_ref, is_ref, r_ref,
                  a_ref, b_ref, rhat_ref, ar_ref, loss_ref):
    p = th_ref[...]
    q = ps_ref[...]
    m = 0.5 * (p + q)
    lm = jnp.log(m)
    t = p * (jnp.log(p) - lm) + q * (jnp.log(q) - lm)
    kl = jnp.sum(t, axis=-1)
    s_uia = 1.0 - 0.5 * kl

    u = u_ref[...]
    i = i_ref[...]
    a = a_ref[...]
    ar = jnp.dot(u * i, (a * a).T, preferred_element_type=F32)
    a_hat = s_uia * ar
    ar_ref[...] = a_hat

    us = us_ref[...]
    isv = is_ref[...]
    pi = us[:, 0:1]
    bu = us[:, 1]
    lu = us[:, 2:10]
    bi = isv[:, 0]
    li = isv[:, 1:9]
    p_uia = pi * lu + (1.0 - pi) * li
    rhat = jnp.sum(p_uia * a_hat, axis=1) + bu + bi + b_ref[0]
    rhat_ref[...] = rhat

    res = r_ref[...] - rhat
    part = 0.5 * jnp.sum(res * res)
    part += 0.5 * L_U * jnp.sum(u * u)
    part += 0.5 * L_I * jnp.sum(i * i)
    part += 0.5 * L_B * (jnp.sum(bu * bu) + jnp.sum(bi * bi))

    @pl.when(pl.program_id(0) == 0)
    def _init():
        loss_ref[0] = 0.5 * L_A * jnp.sum(jnp.abs(a))

    loss_ref[0] += part


_tc_math = pl.pallas_call(
    _tc_math_body,
    grid=(NBLK,),
    in_specs=[
        pl.BlockSpec((BLK, NA, NT), lambda b: (b, 0, 0)),
        pl.BlockSpec((BLK, NA, NT), lambda b: (b, 0, 0)),
        pl.BlockSpec((BLK, NF), lambda b: (b, 0)),
        pl.BlockSpec((BLK, NF), lambda b: (b, 0)),
        pl.BlockSpec((BLK, 16), lambda b: (b, 0)),
        pl.BlockSpec((BLK, 16), lambda b: (b, 0)),
        pl.BlockSpec((BLK,), lambda b: (b,)),
        pl.BlockSpec((NA, NF), lambda b: (0, 0)),
        pl.BlockSpec((1,), lambda b: (0,)),
    ],
    out_specs=[
        pl.BlockSpec((BLK,), lambda b: (b,)),
        pl.BlockSpec((BLK, NA), lambda b: (b, 0)),
        pl.BlockSpec(memory_space=pltpu.SMEM),
    ],
    out_shape=[
        jax.ShapeDtypeStruct((BATCH,), F32),
        jax.ShapeDtypeStruct((BATCH, NA), F32),
        jax.ShapeDtypeStruct((1,), F32),
    ],
)


def kernel(U_ids, I_ids, R, user_table, item_table, Theta_u, Psi_i, Pi_u,
           Lambda_u, Lambda_i, A, Bu, Bi, B):
    zu = jnp.zeros((NU, 6), F32)
    usm = jnp.concatenate([Pi_u[:, None], Bu[:, None], Lambda_u, zu], axis=1)
    zi = jnp.zeros((NI, 7), F32)
    ism = jnp.concatenate([Bi[:, None], Lambda_i, zi], axis=1)

    th_b, ps_b = _sc_theta()(U_ids, I_ids, Theta_u, Psi_i)
    u_b, i_b, us_b, is_b = _sc_small()(
        U_ids, I_ids, user_table, item_table, usm, ism)

    rhat, a_hat, loss = _tc_math(th_b, ps_b, u_b, i_b, us_b, is_b, R, A, B)
    return rhat, a_hat, loss[0]


# speedup vs baseline: 1.1645x; 1.1645x over previous
import functools

import jax
import jax.numpy as jnp
from jax import lax
from jax.experimental import pallas as pl
from jax.experimental.pallas import tpu as pltpu
from jax.experimental.pallas import tpu_sc as plsc

F32 = jnp.float32
I32 = jnp.int32

NU = 100000
NI = 100000
NF = 16
NA = 8
NT = 32
BATCH = 16384
TW = NA * NT

L_U = 0.01
L_I = 0.01
L_A = 0.001
L_B = 0.01

NC = 2
NS = 16
NW = NC * NS
BPW = BATCH // NW
CHC = 64
NCHK = BPW // CHC
LN2 = 0.6931471805599453
SQRTH = 1.41421356


def _ln(x):
    bits = plsc.bitcast(x, I32)
    e = ((bits >> 23) & 0xFF) - 127
    m = plsc.bitcast((bits & 0x007FFFFF) | 0x3F800000, F32)
    big = m > SQRTH
    m = jnp.where(big, 0.5 * m, m)
    ef = (e + big.astype(I32)).astype(F32)
    z = m - 1.0
    y = z * z
    r = 7.0376836292e-2
    for coef in (-1.1514610310e-1, 1.1676998740e-1, -1.2420140846e-1,
                 1.4249322787e-1, -1.6668057665e-1, 2.0000714765e-1,
                 -2.4999993993e-1, 3.3333331174e-1):
        r = r * z + coef
    r = r * z * y - 0.5 * y
    return z + r + LN2 * ef


def _fused_body(uids, iids, rp, th, ps, ut, it, usm, ism, a_hbm,
                out_rhat, out_ahat, out_lp,
                uidx, iidx, thbuf, psbuf, ubuf, ibuf, usbuf, isbuf,
                av, rbuf, rhatb, ahatb, lbuf, sem_s, sem_g):
    wid = lax.axis_index("s") * NC + lax.axis_index("c")
    base = wid * BPW
    pltpu.sync_copy(uids.at[pl.ds(base, BPW)], uidx)
    pltpu.sync_copy(iids.at[pl.ds(base, BPW)], iidx)
    pltpu.sync_copy(a_hbm, av)
    pltpu.sync_copy(rp.at[pl.ds(base, BPW)], rbuf)
    small_cps = [pltpu.async_copy(ut.at[uidx], ubuf, sem_s),
                 pltpu.async_copy(it.at[iidx], ibuf, sem_s),
                 pltpu.async_copy(usm.at[uidx], usbuf, sem_s),
                 pltpu.async_copy(ism.at[iidx], isbuf, sem_s)]
    for cp in small_cps:
        cp.wait()

    a2 = [av[k, :] * av[k, :] for k in range(NA)]
    iota = lax.broadcasted_iota(I32, (16,), 0)
    z16 = jnp.zeros((16,), I32)

    def chunk_body(c, lacc):
        tcp = pltpu.async_copy(th.at[uidx.at[pl.ds(c * CHC, CHC)]], thbuf,
                               sem_g)
        pcp = pltpu.async_copy(ps.at[iidx.at[pl.ds(c * CHC, CHC)]], psbuf,
                               sem_g)
        tcp.wait()
        pcp.wait()

        def group_body(g, lacc2):
            rl = g * 16 + iota
            rw = c * CHC + rl
            start = c * CHC + g * 16
            prod = []
            su2 = jnp.zeros((16,), F32)
            si2 = jnp.zeros((16,), F32)
            for f in range(NF):
                fv = z16 + f
                ufv = plsc.load_gather(ubuf, [rw, fv])
                ifv = plsc.load_gather(ibuf, [rw, fv])
                prod.append(ufv * ifv)
                su2 = su2 + ufv * ufv
                si2 = si2 + ifv * ifv
            pi = plsc.load_gather(usbuf, [rw, z16])
            buv = plsc.load_gather(usbuf, [rw, z16 + 1])
            biv = plsc.load_gather(isbuf, [rw, z16])
            rhat = buv + biv
            for a in range(NA):
                def t_body(t, acc):
                    accp, accq = acc
                    pos = z16 + (a * NT) + t
                    pv = plsc.load_gather(thbuf, [rl, pos])
                    qv = plsc.load_gather(psbuf, [rl, pos])
                    xv = (pv + pv) / (pv + qv)
                    accp = accp + pv * _ln(xv)
                    accq = accq + qv * _ln(2.0 - xv)
                    return (accp, accq)
                accp, accq = lax.fori_loop(
                    0, NT, t_body,
                    (jnp.zeros((16,), F32), jnp.zeros((16,), F32)),
                    unroll=4)
                s_uia = 1.0 - 0.5 * (accp + accq)
                ar = jnp.zeros((16,), F32)
                for f in range(NF):
                    ar = ar + a2[a][f] * prod[f]
                ah = s_uia * ar
                plsc.store_scatter(ahatb, [rw, z16 + a], ah)
                lu = plsc.load_gather(usbuf, [rw, z16 + (2 + a)])
                li = plsc.load_gather(isbuf, [rw, z16 + (1 + a)])
                rhat = rhat + (pi * lu + (1.0 - pi) * li) * ah
            rhatb[pl.ds(start, 16)] = rhat
            res = rbuf[pl.ds(start, 16)] - rhat
            return (lacc2 + 0.5 * (res * res)
                    + (0.5 * L_U) * su2 + (0.5 * L_I) * si2
                    + (0.5 * L_B) * (buv * buv + biv * biv))

        return lax.fori_loop(0, CHC // 16, group_body, lacc)

    lacc = lax.fori_loop(0, NCHK, chunk_body, jnp.zeros((16,), F32))
    labs = jnp.zeros((16,), F32)
    for k in range(NA):
        labs = labs + jnp.abs(av[k, :])
    lacc = lacc + jnp.where(wid == 0, 0.5 * L_A, 0.0) * labs
    lbuf[...] = lacc
    pltpu.sync_copy(rhatb, out_rhat.at[pl.ds(base, BPW)])
    pltpu.sync_copy(ahatb, out_ahat.at[pl.ds(base, BPW)])
    pltpu.sync_copy(lbuf, out_lp.at[wid])


@functools.cache
def _sc_fused():
    return pl.kernel(
        _fused_body,
        out_type=[
            jax.ShapeDtypeStruct((BATCH,), F32),
            jax.ShapeDtypeStruct((BATCH, NA), F32),
            jax.ShapeDtypeStruct((NW, 16), F32),
        ],
        mesh=plsc.VectorSubcoreMesh(core_axis_name="c", subcore_axis_name="s"),
        compiler_params=pltpu.CompilerParams(use_tc_tiling_on_sc=False,
                                             needs_layout_passes=False),
        scratch_types=[
            pltpu.VMEM((BPW,), I32),
            pltpu.VMEM((BPW,), I32),
            pltpu.VMEM((CHC, TW), F32),
            pltpu.VMEM((CHC, TW), F32),
            pltpu.VMEM((BPW, NF), F32),
            pltpu.VMEM((BPW, NF), F32),
            pltpu.VMEM((BPW, 16), F32),
            pltpu.VMEM((BPW, 16), F32),
            pltpu.VMEM((NA, NF), F32),
            pltpu.VMEM((BPW,), F32),
            pltpu.VMEM((BPW,), F32),
            pltpu.VMEM((BPW, NA), F32),
            pltpu.VMEM((16,), F32),
            pltpu.SemaphoreType.DMA,
            pltpu.SemaphoreType.DMA,
        ],
    )


def kernel(U_ids, I_ids, R, user_table, item_table, Theta_u, Psi_i, Pi_u,
           Lambda_u, Lambda_i, A, Bu, Bi, B):
    th_flat = Theta_u.reshape(NU, TW)
    ps_flat = Psi_i.reshape(NI, TW)
    zu = jnp.zeros((NU, 6), F32)
    usm = jnp.concatenate([Pi_u[:, None], Bu[:, None], Lambda_u, zu], axis=1)
    zi = jnp.zeros((NI, 7), F32)
    ism = jnp.concatenate([Bi[:, None], Lambda_i, zi], axis=1)
    rp = R - B[0]

    rhat0, a_hat, lparts = _sc_fused()(
        U_ids, I_ids, rp, th_flat, ps_flat, user_table, item_table,
        usm, ism, A)
    return rhat0 + B[0], a_hat, jnp.sum(lparts)


# speedup vs baseline: 1.8095x; 1.5540x over previous
import functools

import jax
import jax.numpy as jnp
from jax import lax
from jax.experimental import pallas as pl
from jax.experimental.pallas import tpu as pltpu
from jax.experimental.pallas import tpu_sc as plsc

F32 = jnp.float32

NU = 100000
NI = 100000
NF = 16
NA = 8
NT = 32
BATCH = 16384
TW = NA * NT

L_U = 0.01
L_I = 0.01
L_A = 0.001
L_B = 0.01

NC = 2
NS = 16
NW = NC * NS
BPW = BATCH // NW
CH = 64
NCH = BPW // CH


def _gather_wide_body(ids, tab, out, idx, buf, sg0, sg1, sw0, sw1):
    wid = lax.axis_index("s") * NC + lax.axis_index("c")
    base = wid * BPW
    pltpu.sync_copy(ids.at[pl.ds(base, BPW)], idx)

    gsem = (sg0, sg1)
    wsem = (sw0, sw1)
    gp = [None, None]
    wp = [None, None]

    def issue(c):
        b = c % 2
        if wp[b] is not None:
            wp[b].wait()
            wp[b] = None
        sl = pl.ds(c * CH, CH)
        gp[b] = pltpu.async_copy(tab.at[idx.at[sl]], buf.at[b], gsem[b])

    issue(0)
    for c in range(NCH):
        b = c % 2
        if c + 1 < NCH:
            issue(c + 1)
        gp[b].wait()
        osl = pl.ds(base + c * CH, CH)
        wp[b] = pltpu.async_copy(buf.at[b], out.at[osl], wsem[b])
    for b in range(2):
        if wp[b] is not None:
            wp[b].wait()


def _make_wide(tiled):
    params = None if tiled else pltpu.CompilerParams(use_tc_tiling_on_sc=False)
    return pl.kernel(
        _gather_wide_body,
        out_type=[jax.ShapeDtypeStruct((BATCH, TW), F32)],
        mesh=plsc.VectorSubcoreMesh(core_axis_name="c", subcore_axis_name="s"),
        compiler_params=params,
        scratch_types=[
            pltpu.VMEM((BPW,), jnp.int32),
            pltpu.VMEM((2, CH, TW), F32),
            pltpu.SemaphoreType.DMA,
            pltpu.SemaphoreType.DMA,
            pltpu.SemaphoreType.DMA,
            pltpu.SemaphoreType.DMA,
        ],
    )


@functools.cache
def _sc_theta():
    return _make_wide(tiled=True)


@functools.cache
def _sc_psi():
    return _make_wide(tiled=False)


def _sc_small_body(uids, iids, ut, it, usm, ism,
                   out_u, out_i, out_us, out_is,
                   uidx, iidx, ubuf, ibuf, usbuf, isbuf, sem):
    wid = lax.axis_index("s") * NC + lax.axis_index("c")
    base = wid * BPW
    pltpu.sync_copy(uids.at[pl.ds(base, BPW)], uidx)
    pltpu.sync_copy(iids.at[pl.ds(base, BPW)], iidx)
    cps = [pltpu.async_copy(ut.at[uidx], ubuf, sem),
           pltpu.async_copy(it.at[iidx], ibuf, sem),
           pltpu.async_copy(usm.at[uidx], usbuf, sem),
           pltpu.async_copy(ism.at[iidx], isbuf, sem)]
    for cp in cps:
        cp.wait()
    osl = pl.ds(base, BPW)
    pltpu.sync_copy(ubuf, out_u.at[osl])
    pltpu.sync_copy(ibuf, out_i.at[osl])
    pltpu.sync_copy(usbuf, out_us.at[osl])
    pltpu.sync_copy(isbuf, out_is.at[osl])


@functools.cache
def _sc_small():
    return pl.kernel(
        _sc_small_body,
        out_type=[
            jax.ShapeDtypeStruct((BATCH, NF), F32),
            jax.ShapeDtypeStruct((BATCH, NF), F32),
            jax.ShapeDtypeStruct((BATCH, 16), F32),
            jax.ShapeDtypeStruct((BATCH, 16), F32),
        ],
        mesh=plsc.VectorSubcoreMesh(core_axis_name="c", subcore_axis_name="s"),
        compiler_params=pltpu.CompilerParams(use_tc_tiling_on_sc=False),
        scratch_types=[
            pltpu.VMEM((BPW,), jnp.int32),
            pltpu.VMEM((BPW,), jnp.int32),
            pltpu.VMEM((BPW, NF), F32),
            pltpu.VMEM((BPW, NF), F32),
            pltpu.VMEM((BPW, 16), F32),
            pltpu.VMEM((BPW, 16), F32),
            pltpu.SemaphoreType.DMA,
        ],
    )


BLK = 1024
NBLK = BATCH // BLK


def _tc_math_body(th_ref, ps_ref, u_ref, i_ref, us_ref, is_ref, r_ref,
                  a_ref, b_ref, rhat_ref, ar_ref, loss_ref):
    p = th_ref[...]
    q = ps_ref[...]
    m = 0.5 * (p + q)
    lm = jnp.log(m)
    t = p * (jnp.log(p) - lm) + q * (jnp.log(q) - lm)
    asp = lax.broadcasted_iota(jnp.int32, (TW, NA), 0) // NT
    e = (asp == lax.broadcasted_iota(jnp.int32, (TW, NA), 1)).astype(F32)
    kl = jnp.dot(t, e, preferred_element_type=F32)
    s_uia = 1.0 - 0.5 * kl

    u = u_ref[...]
    i = i_ref[...]
    a = a_ref[...]
    ar = jnp.dot(u * i, (a * a).T, preferred_element_type=F32)
    a_hat = s_uia * ar
    ar_ref[...] = a_hat

    us = us_ref[...]
    isv = is_ref[...]
    pi = us[:, 0:1]
    bu = us[:, 1]
    lu = us[:, 2:10]
    bi = isv[:, 0]
    li = isv[:, 1:9]
    p_uia = pi * lu + (1.0 - pi) * li
    rhat = jnp.sum(p_uia * a_hat, axis=1) + bu + bi + b_ref[0]
    rhat_ref[...] = rhat

    res = r_ref[...] - rhat
    part = 0.5 * jnp.sum(res * res)
    part += 0.5 * L_U * jnp.sum(u * u)
    part += 0.5 * L_I * jnp.sum(i * i)
    part += 0.5 * L_B * (jnp.sum(bu * bu) + jnp.sum(bi * bi))

    @pl.when(pl.program_id(0) == 0)
    def _init():
        loss_ref[0] = 0.5 * L_A * jnp.sum(jnp.abs(a))

    loss_ref[0] += part


_tc_math = pl.pallas_call(
    _tc_math_body,
    grid=(NBLK,),
    in_specs=[
        pl.BlockSpec((BLK, TW), lambda b: (b, 0)),
        pl.BlockSpec((BLK, TW), lambda b: (b, 0)),
        pl.BlockSpec((BLK, NF), lambda b: (b, 0)),
        pl.BlockSpec((BLK, NF), lambda b: (b, 0)),
        pl.BlockSpec((BLK, 16), lambda b: (b, 0)),
        pl.BlockSpec((BLK, 16), lambda b: (b, 0)),
        pl.BlockSpec((BLK,), lambda b: (b,)),
        pl.BlockSpec((NA, NF), lambda b: (0, 0)),
        pl.BlockSpec((1,), lambda b: (0,)),
    ],
    out_specs=[
        pl.BlockSpec((BLK,), lambda b: (b,)),
        pl.BlockSpec((BLK, NA), lambda b: (b, 0)),
        pl.BlockSpec(memory_space=pltpu.SMEM),
    ],
    out_shape=[
        jax.ShapeDtypeStruct((BATCH,), F32),
        jax.ShapeDtypeStruct((BATCH, NA), F32),
        jax.ShapeDtypeStruct((1,), F32),
    ],
)


def kernel(U_ids, I_ids, R, user_table, item_table, Theta_u, Psi_i, Pi_u,
           Lambda_u, Lambda_i, A, Bu, Bi, B):
    th_flat = Theta_u.reshape(NU, TW)
    ps_flat = Psi_i.reshape(NI, TW)
    zu = jnp.zeros((NU, 6), F32)
    usm = jnp.concatenate([Pi_u[:, None], Bu[:, None], Lambda_u, zu], axis=1)
    zi = jnp.zeros((NI, 7), F32)
    ism = jnp.concatenate([Bi[:, None], Lambda_i, zi], axis=1)

    (ps_b,) = _sc_psi()(I_ids, ps_flat)
    u_b, i_b, us_b, is_b = _sc_small()(
        U_ids, I_ids, user_table, item_table, usm, ism)
    (th_b,) = _sc_theta()(U_ids, th_flat)

    rhat, a_hat, loss = _tc_math(th_b, ps_b, u_b, i_b, us_b, is_b, R, A, B)
    return rhat, a_hat, loss[0]


# speedup vs baseline: 2.0320x; 1.1229x over previous
import functools

import jax
import jax.numpy as jnp
from jax import lax
from jax.experimental import pallas as pl
from jax.experimental.pallas import tpu as pltpu
from jax.experimental.pallas import tpu_sc as plsc

F32 = jnp.float32

NU = 100000
NI = 100000
NF = 16
NA = 8
NT = 32
BATCH = 16384
TW = NA * NT

L_U = 0.01
L_I = 0.01
L_A = 0.001
L_B = 0.01

NC = 2
NS = 16
NW = NC * NS
BPW = BATCH // NW
CH = 64
NCH = BPW // CH


def _sc_theta_body(uids, iids, th, ps, out_th, out_ps,
                   uidx, iidx, thbuf, psbuf, sg0, sg1, sw0, sw1):
    wid = lax.axis_index("s") * NC + lax.axis_index("c")
    base = wid * BPW
    pltpu.sync_copy(uids.at[pl.ds(base, BPW)], uidx)
    pltpu.sync_copy(iids.at[pl.ds(base, BPW)], iidx)

    gsem = (sg0, sg1)
    wsem = (sw0, sw1)
    gp = [None, None]
    wp = [None, None]

    def issue(c):
        b = c % 2
        if wp[b] is not None:
            for cp in wp[b]:
                cp.wait()
            wp[b] = None
        sl = pl.ds(c * CH, CH)
        gp[b] = [pltpu.async_copy(th.at[uidx.at[sl]], thbuf.at[b], gsem[b]),
                 pltpu.async_copy(ps.at[iidx.at[sl]], psbuf.at[b], gsem[b])]

    issue(0)
    for c in range(NCH):
        b = c % 2
        if c + 1 < NCH:
            issue(c + 1)
        for cp in gp[b]:
            cp.wait()
        osl = pl.ds(base + c * CH, CH)
        wp[b] = [pltpu.async_copy(thbuf.at[b], out_th.at[osl], wsem[b]),
                 pltpu.async_copy(psbuf.at[b], out_ps.at[osl], wsem[b])]
    for b in range(2):
        if wp[b] is not None:
            for cp in wp[b]:
                cp.wait()


@functools.cache
def _sc_theta():
    return pl.kernel(
        _sc_theta_body,
        out_type=[
            jax.ShapeDtypeStruct((BATCH, TW), F32),
            jax.ShapeDtypeStruct((BATCH, TW), F32),
        ],
        mesh=plsc.VectorSubcoreMesh(core_axis_name="c", subcore_axis_name="s"),
        scratch_types=[
            pltpu.VMEM((BPW,), jnp.int32),
            pltpu.VMEM((BPW,), jnp.int32),
            pltpu.VMEM((2, CH, TW), F32),
            pltpu.VMEM((2, CH, TW), F32),
            pltpu.SemaphoreType.DMA,
            pltpu.SemaphoreType.DMA,
            pltpu.SemaphoreType.DMA,
            pltpu.SemaphoreType.DMA,
        ],
    )


def _sc_small_body(uids, iids, ut, it, usm, ism,
                   out_u, out_i, out_us, out_is,
                   uidx, iidx, ubuf, ibuf, usbuf, isbuf, sem):
    wid = lax.axis_index("s") * NC + lax.axis_index("c")
    base = wid * BPW
    pltpu.sync_copy(uids.at[pl.ds(base, BPW)], uidx)
    pltpu.sync_copy(iids.at[pl.ds(base, BPW)], iidx)
    cps = [pltpu.async_copy(ut.at[uidx], ubuf, sem),
           pltpu.async_copy(it.at[iidx], ibuf, sem),
           pltpu.async_copy(usm.at[uidx], usbuf, sem),
           pltpu.async_copy(ism.at[iidx], isbuf, sem)]
    for cp in cps:
        cp.wait()
    osl = pl.ds(base, BPW)
    pltpu.sync_copy(ubuf, out_u.at[osl])
    pltpu.sync_copy(ibuf, out_i.at[osl])
    pltpu.sync_copy(usbuf, out_us.at[osl])
    pltpu.sync_copy(isbuf, out_is.at[osl])


@functools.cache
def _sc_small():
    return pl.kernel(
        _sc_small_body,
        out_type=[
            jax.ShapeDtypeStruct((BATCH, NF), F32),
            jax.ShapeDtypeStruct((BATCH, NF), F32),
            jax.ShapeDtypeStruct((BATCH, 16), F32),
            jax.ShapeDtypeStruct((BATCH, 16), F32),
        ],
        mesh=plsc.VectorSubcoreMesh(core_axis_name="c", subcore_axis_name="s"),
        compiler_params=pltpu.CompilerParams(use_tc_tiling_on_sc=False),
        scratch_types=[
            pltpu.VMEM((BPW,), jnp.int32),
            pltpu.VMEM((BPW,), jnp.int32),
            pltpu.VMEM((BPW, NF), F32),
            pltpu.VMEM((BPW, NF), F32),
            pltpu.VMEM((BPW, 16), F32),
            pltpu.VMEM((BPW, 16), F32),
            pltpu.SemaphoreType.DMA,
        ],
    )


BLK = 1024
NBLK = BATCH // BLK


def _tc_math_body(th_ref, ps_ref, u_ref, i_ref, us_ref, is_ref, r_ref,
                  a_ref, b_ref, rhat_ref, ar_ref, loss_ref):
    p = th_ref[...]
    q = ps_ref[...]
    m = 0.5 * (p + q)
    lm = jnp.log(m)
    t = p * (jnp.log(p) - lm) + q * (jnp.log(q) - lm)
    asp = lax.broadcasted_iota(jnp.int32, (TW, NA), 0) // NT
    e = (asp == lax.broadcasted_iota(jnp.int32, (TW, NA), 1)).astype(F32)
    kl = jnp.dot(t, e, preferred_element_type=F32)
    s_uia = 1.0 - 0.5 * kl

    u = u_ref[...]
    i = i_ref[...]
    a = a_ref[...]
    ar = jnp.dot(u * i, (a * a).T, preferred_element_type=F32)
    a_hat = s_uia * ar
    ar_ref[...] = a_hat

    us = us_ref[...]
    isv = is_ref[...]
    pi = us[:, 0:1]
    bu = us[:, 1]
    lu = us[:, 2:10]
    bi = isv[:, 0]
    li = isv[:, 1:9]
    p_uia = pi * lu + (1.0 - pi) * li
    rhat = jnp.sum(p_uia * a_hat, axis=1) + bu + bi + b_ref[0]
    rhat_ref[...] = rhat

    res = r_ref[...] - rhat
    part = 0.5 * jnp.sum(res * res)
    part += 0.5 * L_U * jnp.sum(u * u)
    part += 0.5 * L_I * jnp.sum(i * i)
    part += 0.5 * L_B * (jnp.sum(bu * bu) + jnp.sum(bi * bi))

    @pl.when(pl.program_id(0) == 0)
    def _init():
        loss_ref[0] = 0.5 * L_A * jnp.sum(jnp.abs(a))

    loss_ref[0] += part


_tc_math = pl.pallas_call(
    _tc_math_body,
    grid=(NBLK,),
    in_specs=[
        pl.BlockSpec((BLK, TW), lambda b: (b, 0)),
        pl.BlockSpec((BLK, TW), lambda b: (b, 0)),
        pl.BlockSpec((BLK, NF), lambda b: (b, 0)),
        pl.BlockSpec((BLK, NF), lambda b: (b, 0)),
        pl.BlockSpec((BLK, 16), lambda b: (b, 0)),
        pl.BlockSpec((BLK, 16), lambda b: (b, 0)),
        pl.BlockSpec((BLK,), lambda b: (b,)),
        pl.BlockSpec((NA, NF), lambda b: (0, 0)),
        pl.BlockSpec((1,), lambda b: (0,)),
    ],
    out_specs=[
        pl.BlockSpec((BLK,), lambda b: (b,)),
        pl.BlockSpec((BLK, NA), lambda b: (b, 0)),
        pl.BlockSpec(memory_space=pltpu.SMEM),
    ],
    out_shape=[
        jax.ShapeDtypeStruct((BATCH,), F32),
        jax.ShapeDtypeStruct((BATCH, NA), F32),
        jax.ShapeDtypeStruct((1,), F32),
    ],
)


def kernel(U_ids, I_ids, R, user_table, item_table, Theta_u, Psi_i, Pi_u,
           Lambda_u, Lambda_i, A, Bu, Bi, B):
    th_flat = Theta_u.reshape(NU, TW)
    ps_flat = Psi_i.reshape(NI, TW)
    zu = jnp.zeros((NU, 6), F32)
    usm = jnp.concatenate([Pi_u[:, None], Bu[:, None], Lambda_u, zu], axis=1)
    zi = jnp.zeros((NI, 7), F32)
    ism = jnp.concatenate([Bi[:, None], Lambda_i, zi], axis=1)

    th_b, ps_b = _sc_theta()(U_ids, I_ids, th_flat, ps_flat)
    u_b, i_b, us_b, is_b = _sc_small()(
        U_ids, I_ids, user_table, item_table, usm, ism)

    rhat, a_hat, loss = _tc_math(th_b, ps_b, u_b, i_b, us_b, is_b, R, A, B)
    return rhat, a_hat, loss[0]


# speedup vs baseline: 2.0656x; 1.0166x over previous
import functools

import jax
import jax.numpy as jnp
from jax import lax
from jax.experimental import pallas as pl
from jax.experimental.pallas import tpu as pltpu
from jax.experimental.pallas import tpu_sc as plsc

F32 = jnp.float32

NU = 100000
NI = 100000
NF = 16
NA = 8
NT = 32
BATCH = 16384
TW = NA * NT

L_U = 0.01
L_I = 0.01
L_A = 0.001
L_B = 0.01

NC = 2
NS = 16
NW = NC * NS
BPW = BATCH // NW
CH = 64
NCH = BPW // CH


def _sc_theta_body(uids, iids, th, ps, out_th, out_ps,
                   uidx, iidx, thbuf, psbuf, sg0, sg1, sw0, sw1):
    wid = lax.axis_index("s") * NC + lax.axis_index("c")
    base = wid * BPW
    pltpu.sync_copy(uids.at[pl.ds(base, BPW)], uidx)
    pltpu.sync_copy(iids.at[pl.ds(base, BPW)], iidx)

    gsem = (sg0, sg1)
    wsem = (sw0, sw1)
    gp = [None, None]
    wp = [None, None]

    def issue(c):
        b = c % 2
        if wp[b] is not None:
            for cp in wp[b]:
                cp.wait()
            wp[b] = None
        sl = pl.ds(c * CH, CH)
        gp[b] = [pltpu.async_copy(th.at[uidx.at[sl]], thbuf.at[b], gsem[b]),
                 pltpu.async_copy(ps.at[iidx.at[sl]], psbuf.at[b], gsem[b])]

    issue(0)
    for c in range(NCH):
        b = c % 2
        if c + 1 < NCH:
            issue(c + 1)
        for cp in gp[b]:
            cp.wait()
        osl = pl.ds(base + c * CH, CH)
        wp[b] = [pltpu.async_copy(thbuf.at[b], out_th.at[osl], wsem[b]),
                 pltpu.async_copy(psbuf.at[b], out_ps.at[osl], wsem[b])]
    for b in range(2):
        if wp[b] is not None:
            for cp in wp[b]:
                cp.wait()


@functools.cache
def _sc_theta():
    return pl.kernel(
        _sc_theta_body,
        out_type=[
            jax.ShapeDtypeStruct((BATCH, TW), F32),
            jax.ShapeDtypeStruct((BATCH, TW), F32),
        ],
        mesh=plsc.VectorSubcoreMesh(core_axis_name="c", subcore_axis_name="s"),
        scratch_types=[
            pltpu.VMEM((BPW,), jnp.int32),
            pltpu.VMEM((BPW,), jnp.int32),
            pltpu.VMEM((2, CH, TW), F32),
            pltpu.VMEM((2, CH, TW), F32),
            pltpu.SemaphoreType.DMA,
            pltpu.SemaphoreType.DMA,
            pltpu.SemaphoreType.DMA,
            pltpu.SemaphoreType.DMA,
        ],
    )


def _sc_small_body(uids, iids, usm, ism,
                   out_us, out_is,
                   uidx, iidx, usbuf, isbuf, sem):
    wid = lax.axis_index("s") * NC + lax.axis_index("c")
    base = wid * BPW
    pltpu.sync_copy(uids.at[pl.ds(base, BPW)], uidx)
    pltpu.sync_copy(iids.at[pl.ds(base, BPW)], iidx)
    cps = [pltpu.async_copy(usm.at[uidx], usbuf, sem),
           pltpu.async_copy(ism.at[iidx], isbuf, sem)]
    for cp in cps:
        cp.wait()
    osl = pl.ds(base, BPW)
    pltpu.sync_copy(usbuf, out_us.at[osl])
    pltpu.sync_copy(isbuf, out_is.at[osl])


@functools.cache
def _sc_small():
    return pl.kernel(
        _sc_small_body,
        out_type=[
            jax.ShapeDtypeStruct((BATCH, 32), F32),
            jax.ShapeDtypeStruct((BATCH, 32), F32),
        ],
        mesh=plsc.VectorSubcoreMesh(core_axis_name="c", subcore_axis_name="s"),
        compiler_params=pltpu.CompilerParams(use_tc_tiling_on_sc=False),
        scratch_types=[
            pltpu.VMEM((BPW,), jnp.int32),
            pltpu.VMEM((BPW,), jnp.int32),
            pltpu.VMEM((BPW, 32), F32),
            pltpu.VMEM((BPW, 32), F32),
            pltpu.SemaphoreType.DMA,
        ],
    )


BLK = 1024
NBLK = BATCH // BLK


def _tc_math_body(th_ref, ps_ref, us_ref, is_ref, r_ref,
                  a_ref, b_ref, rhat_ref, ar_ref, loss_ref):
    p = th_ref[...]
    q = ps_ref[...]
    x = (p + p) / (p + q)
    t = p * jnp.log(x) + q * jnp.log(2.0 - x)
    asp = lax.broadcasted_iota(jnp.int32, (TW, NA), 0) // NT
    e = (asp == lax.broadcasted_iota(jnp.int32, (TW, NA), 1)).astype(F32)
    kl = jnp.dot(t, e, preferred_element_type=F32)
    s_uia = 1.0 - 0.5 * kl

    us = us_ref[...]
    isv = is_ref[...]
    u = us[:, 0:NF]
    i = isv[:, 0:NF]
    a = a_ref[...]
    ar = jnp.dot(u * i, (a * a).T, preferred_element_type=F32)
    a_hat = s_uia * ar
    ar_ref[...] = a_hat

    pi = us[:, 16:17]
    bu = us[:, 17]
    lu = us[:, 18:26]
    bi = isv[:, 16]
    li = isv[:, 17:25]
    p_uia = pi * lu + (1.0 - pi) * li
    rhat = jnp.sum(p_uia * a_hat, axis=1) + bu + bi + b_ref[0]
    rhat_ref[...] = rhat

    res = r_ref[...] - rhat
    part = 0.5 * jnp.sum(res * res)
    part += 0.5 * L_U * jnp.sum(u * u)
    part += 0.5 * L_I * jnp.sum(i * i)
    part += 0.5 * L_B * (jnp.sum(bu * bu) + jnp.sum(bi * bi))

    @pl.when(pl.program_id(0) == 0)
    def _init():
        loss_ref[0] = 0.5 * L_A * jnp.sum(jnp.abs(a))

    loss_ref[0] += part


_tc_math = pl.pallas_call(
    _tc_math_body,
    grid=(NBLK,),
    in_specs=[
        pl.BlockSpec((BLK, TW), lambda b: (b, 0)),
        pl.BlockSpec((BLK, TW), lambda b: (b, 0)),
        pl.BlockSpec((BLK, 32), lambda b: (b, 0)),
        pl.BlockSpec((BLK, 32), lambda b: (b, 0)),
        pl.BlockSpec((BLK,), lambda b: (b,)),
        pl.BlockSpec((NA, NF), lambda b: (0, 0)),
        pl.BlockSpec((1,), lambda b: (0,)),
    ],
    out_specs=[
        pl.BlockSpec((BLK,), lambda b: (b,)),
        pl.BlockSpec((BLK, NA), lambda b: (b, 0)),
        pl.BlockSpec(memory_space=pltpu.SMEM),
    ],
    out_shape=[
        jax.ShapeDtypeStruct((BATCH,), F32),
        jax.ShapeDtypeStruct((BATCH, NA), F32),
        jax.ShapeDtypeStruct((1,), F32),
    ],
)


def kernel(U_ids, I_ids, R, user_table, item_table, Theta_u, Psi_i, Pi_u,
           Lambda_u, Lambda_i, A, Bu, Bi, B):
    th_flat = Theta_u.reshape(NU, TW)
    ps_flat = Psi_i.reshape(NI, TW)
    zu = jnp.zeros((NU, 6), F32)
    usm = jnp.concatenate(
        [user_table, Pi_u[:, None], Bu[:, None], Lambda_u, zu], axis=1)
    zi = jnp.zeros((NI, 7), F32)
    ism = jnp.concatenate([item_table, Bi[:, None], Lambda_i, zi], axis=1)

    th_b, ps_b = _sc_theta()(U_ids, I_ids, th_flat, ps_flat)
    us_b, is_b = _sc_small()(U_ids, I_ids, usm, ism)

    rhat, a_hat, loss = _tc_math(th_b, ps_b, us_b, is_b, R, A, B)
    return rhat, a_hat, loss[0]


# speedup vs baseline: 2.0825x; 1.0082x over previous
import functools

import jax
import jax.numpy as jnp
from jax import lax
from jax.experimental import pallas as pl
from jax.experimental.pallas import tpu as pltpu
from jax.experimental.pallas import tpu_sc as plsc

F32 = jnp.float32

NU = 100000
NI = 100000
NF = 16
NA = 8
NT = 32
BATCH = 16384
TW = NA * NT

L_U = 0.01
L_I = 0.01
L_A = 0.001
L_B = 0.01

NC = 2
NS = 16
NW = NC * NS
BPW = BATCH // NW
CH = 64
NCH = BPW // CH


def _gather_wide_body(ids, tab, out, idx, buf, sg0, sg1, sw0, sw1):
    wid = lax.axis_index("s") * NC + lax.axis_index("c")
    base = wid * BPW
    pltpu.sync_copy(ids.at[pl.ds(base, BPW)], idx)

    gsem = (sg0, sg1)
    wsem = (sw0, sw1)
    gp = [None, None]
    wp = [None, None]

    def issue(c):
        b = c % 2
        if wp[b] is not None:
            wp[b].wait()
            wp[b] = None
        sl = pl.ds(c * CH, CH)
        gp[b] = pltpu.async_copy(tab.at[idx.at[sl]], buf.at[b], gsem[b])

    issue(0)
    for c in range(NCH):
        b = c % 2
        if c + 1 < NCH:
            issue(c + 1)
        gp[b].wait()
        osl = pl.ds(base + c * CH, CH)
        wp[b] = pltpu.async_copy(buf.at[b], out.at[osl], wsem[b])
    for b in range(2):
        if wp[b] is not None:
            wp[b].wait()


@functools.cache
def _sc_wide():
    return pl.kernel(
        _gather_wide_body,
        out_type=[jax.ShapeDtypeStruct((BATCH, TW), F32)],
        mesh=plsc.VectorSubcoreMesh(core_axis_name="c", subcore_axis_name="s"),
        scratch_types=[
            pltpu.VMEM((BPW,), jnp.int32),
            pltpu.VMEM((2, CH, TW), F32),
            pltpu.SemaphoreType.DMA,
            pltpu.SemaphoreType.DMA,
            pltpu.SemaphoreType.DMA,
            pltpu.SemaphoreType.DMA,
        ],
    )


def _sc_small_body(uids, iids, usm, ism,
                   out_us, out_is,
                   uidx, iidx, usbuf, isbuf, sem):
    wid = lax.axis_index("s") * NC + lax.axis_index("c")
    base = wid * BPW
    pltpu.sync_copy(uids.at[pl.ds(base, BPW)], uidx)
    pltpu.sync_copy(iids.at[pl.ds(base, BPW)], iidx)
    cps = [pltpu.async_copy(usm.at[uidx], usbuf, sem),
           pltpu.async_copy(ism.at[iidx], isbuf, sem)]
    for cp in cps:
        cp.wait()
    osl = pl.ds(base, BPW)
    pltpu.sync_copy(usbuf, out_us.at[osl])
    pltpu.sync_copy(isbuf, out_is.at[osl])


@functools.cache
def _sc_small():
    return pl.kernel(
        _sc_small_body,
        out_type=[
            jax.ShapeDtypeStruct((BATCH, 32), F32),
            jax.ShapeDtypeStruct((BATCH, 32), F32),
        ],
        mesh=plsc.VectorSubcoreMesh(core_axis_name="c", subcore_axis_name="s"),
        compiler_params=pltpu.CompilerParams(use_tc_tiling_on_sc=False),
        scratch_types=[
            pltpu.VMEM((BPW,), jnp.int32),
            pltpu.VMEM((BPW,), jnp.int32),
            pltpu.VMEM((BPW, 32), F32),
            pltpu.VMEM((BPW, 32), F32),
            pltpu.SemaphoreType.DMA,
        ],
    )


BLK = 1024
NBLK = BATCH // BLK


def _tc_math_body(th_ref, ps_ref, us_ref, is_ref, r_ref,
                  a_ref, b_ref, rhat_ref, ar_ref, loss_ref):
    p = th_ref[...]
    q = ps_ref[...]
    x = (p + p) / (p + q)
    t = p * jnp.log(x) + q * jnp.log(2.0 - x)
    asp = lax.broadcasted_iota(jnp.int32, (TW, NA), 0) // NT
    e = (asp == lax.broadcasted_iota(jnp.int32, (TW, NA), 1)).astype(F32)
    kl = jnp.dot(t, e, preferred_element_type=F32)
    s_uia = 1.0 - 0.5 * kl

    us = us_ref[...]
    isv = is_ref[...]
    u = us[:, 0:NF]
    i = isv[:, 0:NF]
    a = a_ref[...]
    ar = jnp.dot(u * i, (a * a).T, preferred_element_type=F32)
    a_hat = s_uia * ar
    ar_ref[...] = a_hat

    pi = us[:, 16:17]
    bu = us[:, 17]
    lu = us[:, 18:26]
    bi = isv[:, 16]
    li = isv[:, 17:25]
    p_uia = pi * lu + (1.0 - pi) * li
    rhat = jnp.sum(p_uia * a_hat, axis=1) + bu + bi + b_ref[0]
    rhat_ref[...] = rhat

    res = r_ref[...] - rhat
    part = 0.5 * jnp.sum(res * res)
    part += 0.5 * L_U * jnp.sum(u * u)
    part += 0.5 * L_I * jnp.sum(i * i)
    part += 0.5 * L_B * (jnp.sum(bu * bu) + jnp.sum(bi * bi))

    @pl.when(pl.program_id(0) == 0)
    def _init():
        loss_ref[0] = 0.5 * L_A * jnp.sum(jnp.abs(a))

    loss_ref[0] += part


_tc_math = pl.pallas_call(
    _tc_math_body,
    grid=(NBLK,),
    in_specs=[
        pl.BlockSpec((BLK, TW), lambda b: (b, 0)),
        pl.BlockSpec((BLK, TW), lambda b: (b, 0)),
        pl.BlockSpec((BLK, 32), lambda b: (b, 0)),
        pl.BlockSpec((BLK, 32), lambda b: (b, 0)),
        pl.BlockSpec((BLK,), lambda b: (b,)),
        pl.BlockSpec((NA, NF), lambda b: (0, 0)),
        pl.BlockSpec((1,), lambda b: (0,)),
    ],
    out_specs=[
        pl.BlockSpec((BLK,), lambda b: (b,)),
        pl.BlockSpec((BLK, NA), lambda b: (b, 0)),
        pl.BlockSpec(memory_space=pltpu.SMEM),
    ],
    out_shape=[
        jax.ShapeDtypeStruct((BATCH,), F32),
        jax.ShapeDtypeStruct((BATCH, NA), F32),
        jax.ShapeDtypeStruct((1,), F32),
    ],
)


def kernel(U_ids, I_ids, R, user_table, item_table, Theta_u, Psi_i, Pi_u,
           Lambda_u, Lambda_i, A, Bu, Bi, B):
    th_flat = Theta_u.reshape(NU, TW)
    ps_flat = Psi_i.reshape(NI, TW)
    zu = jnp.zeros((NU, 6), F32)
    usm = jnp.concatenate(
        [user_table, Pi_u[:, None], Bu[:, None], Lambda_u, zu], axis=1)
    zi = jnp.zeros((NI, 7), F32)
    ism = jnp.concatenate([item_table, Bi[:, None], Lambda_i, zi], axis=1)

    (ps_b,) = _sc_wide()(I_ids, ps_flat)
    us_b, is_b = _sc_small()(U_ids, I_ids, usm, ism)
    (th_b,) = _sc_wide()(U_ids, th_flat)

    rhat, a_hat, loss = _tc_math(th_b, ps_b, us_b, is_b, R, A, B)
    return rhat, a_hat, loss[0]
